# Initial kernel scaffold; baseline (speedup 1.0000x reference)
import jax
import jax.numpy as jnp
from jax.experimental import pallas as pl


def kernel(x, edge_index, W1, b1, W2, b2, Wa, ba):
    raise NotImplementedError("write your pallas kernel here")


# speedup vs baseline: 14.6202x; 14.6202x over previous
import functools

import jax
import jax.numpy as jnp
from jax import lax
from jax.experimental import pallas as pl
from jax.experimental.pallas import tpu as pltpu
from jax.experimental.pallas import tpu_sc as plsc

N_NODES = 10000
N_EDGES = 320000
NC = 2
NS = 16
NW = NC * NS
E_PER_TILE = N_EDGES // NW
CHUNK = 80
N_CHUNKS = E_PER_TILE // CHUNK
RPT = 632
N_PAD = RPT * NS


def _make_seg_agg(D, table_is_ones):
    mesh = plsc.VectorSubcoreMesh(core_axis_name="c", subcore_axis_name="s")
    wide = D >= 8

    scratch = [
        pltpu.VMEM((CHUNK,), jnp.int32),
        pltpu.VMEM((CHUNK,), jnp.int32),
        pltpu.VMEM((CHUNK, D), jnp.float32),
        pltpu.VMEM_SHARED((N_PAD, D), jnp.float32),
        pltpu.SemaphoreType.DMA,
    ]

    @functools.partial(
        pl.kernel,
        out_type=jax.ShapeDtypeStruct((NC, N_PAD, D), jnp.float32),
        mesh=mesh,
        scratch_types=scratch,
        compiler_params=pltpu.CompilerParams(use_tc_tiling_on_sc=False),
    )
    def k(table_hbm, src_hbm, dst_hbm, zeros_hbm, out_hbm,
          idx_s, idx_d, rows, acc, sem):
        c = lax.axis_index("c")
        s = lax.axis_index("s")

        if wide:
            pltpu.sync_copy(zeros_hbm, acc.at[pl.ds(s * RPT, RPT)])
        else:
            @pl.when(s == 0)
            def _zero():
                pltpu.sync_copy(zeros_hbm, acc)
        if table_is_ones:
            pltpu.sync_copy(table_hbm, rows)
        plsc.subcore_barrier()

        base = (c * NS + s) * E_PER_TILE

        def body(j, carry):
            off = base + j * CHUNK
            pltpu.sync_copy(dst_hbm.at[pl.ds(off, CHUNK)], idx_d)
            if not table_is_ones:
                pltpu.sync_copy(src_hbm.at[pl.ds(off, CHUNK)], idx_s)
                pltpu.async_copy(table_hbm.at[idx_s], rows, sem).wait()
            pltpu.sync_copy(rows, acc.at[idx_d], add=True)
            return carry

        lax.fori_loop(0, N_CHUNKS, body, 0)

        plsc.subcore_barrier()
        if wide:
            pltpu.sync_copy(acc.at[pl.ds(s * RPT, RPT)],
                            out_hbm.at[c, pl.ds(s * RPT, RPT)])
        else:
            @pl.when(s == 0)
            def _copy_out():
                pltpu.sync_copy(acc, out_hbm.at[c])

    return k


_seg_agg = {D: _make_seg_agg(D, False) for D in (64, 32)}
_deg_agg = _make_seg_agg(8, True)


def _mm_body(x_ref, w_ref, o_ref):
    o_ref[...] = jnp.dot(x_ref[...], w_ref[...],
                         preferred_element_type=jnp.float32)


def _scale_body(h_ref, degp_ref, g_ref, dis_ref):
    deg = degp_ref[0, :, :1] + degp_ref[1, :, :1] + 1.0
    dis = lax.rsqrt(deg)
    dis_ref[...] = dis
    g_ref[...] = h_ref[...] * dis


def _layer_body(accp_ref, g_ref, dis_ref, b_ref, w_ref, g2_ref):
    dis = dis_ref[...]
    h = dis * (accp_ref[0] + accp_ref[1] + g_ref[...]) + b_ref[...]
    h = jnp.maximum(h, 0.0)
    g2_ref[...] = dis * jnp.dot(h, w_ref[...],
                                preferred_element_type=jnp.float32)


def _final_body(accp_ref, g_ref, dis_ref, b_ref, wat_ref, ba_ref, o_ref):
    dis = dis_ref[...]
    h = dis * (accp_ref[0] + accp_ref[1] + g_ref[...]) + b_ref[...]
    h = jnp.maximum(h, 0.0)
    logit = jnp.sum(h * wat_ref[...], axis=1, keepdims=True) + ba_ref[...]
    attn = 1.0 / (1.0 + jnp.exp(-logit))
    o_ref[...] = h * attn


def _tc(body, out_shape, *args):
    return pl.pallas_call(
        body, out_shape=jax.ShapeDtypeStruct(out_shape, jnp.float32))(*args)


def kernel(x, edge_index, W1, b1, W2, b2, Wa, ba):
    src = edge_index[0].astype(jnp.int32)
    dst = edge_index[1].astype(jnp.int32)

    ones_rows = jnp.ones((CHUNK, 8), jnp.float32)
    zeros_rpt8 = jnp.zeros((RPT, 8), jnp.float32)
    zeros_rpt64 = jnp.zeros((RPT, 64), jnp.float32)
    zeros_rpt32 = jnp.zeros((RPT, 32), jnp.float32)

    h1p = _tc(_mm_body, (N_NODES, 64), x, W1)
    degp = _deg_agg(ones_rows, src, dst, zeros_rpt8)[:, :N_NODES]

    g1, dis = pl.pallas_call(
        _scale_body,
        out_shape=(jax.ShapeDtypeStruct((N_NODES, 64), jnp.float32),
                   jax.ShapeDtypeStruct((N_NODES, 1), jnp.float32)),
    )(h1p, degp)

    acc1 = _seg_agg[64](g1, src, dst, zeros_rpt64)[:, :N_NODES]
    g2 = _tc(_layer_body, (N_NODES, 32),
             acc1, g1, dis, b1.reshape(1, 64), W2)

    acc2 = _seg_agg[32](g2, src, dst, zeros_rpt32)[:, :N_NODES]
    out = _tc(_final_body, (N_NODES, 32),
              acc2, g2, dis, b2.reshape(1, 32), Wa.reshape(1, 32),
              ba.reshape(1, 1))
    return out


# speedup vs baseline: 40.8862x; 2.7966x over previous
import functools

import jax
import jax.numpy as jnp
from jax import lax
from jax.experimental import pallas as pl
from jax.experimental.pallas import tpu as pltpu
from jax.experimental.pallas import tpu_sc as plsc

N_NODES = 10000
N_EDGES = 320000
NC = 2
NS = 16
NW = NC * NS
E_PER_TILE = N_EDGES // NW
CH = 125
NCH = E_PER_TILE // CH
RPT = 632
N_PAD = RPT * NS


def _make_seg_agg(D, table_is_ones):
    mesh = plsc.VectorSubcoreMesh(core_axis_name="c", subcore_axis_name="s")

    scratch = [
        pltpu.VMEM((NCH, CH), jnp.int32),
        pltpu.VMEM((NCH, CH), jnp.int32),
        pltpu.VMEM((CH, D), jnp.float32),
        pltpu.VMEM((CH, D), jnp.float32),
        pltpu.VMEM_SHARED((N_PAD, D), jnp.float32),
        pltpu.SemaphoreType.DMA,
        pltpu.SemaphoreType.DMA,
        pltpu.SemaphoreType.DMA,
    ]

    @functools.partial(
        pl.kernel,
        out_type=jax.ShapeDtypeStruct((NC, N_PAD, D), jnp.float32),
        mesh=mesh,
        scratch_types=scratch,
        compiler_params=pltpu.CompilerParams(use_tc_tiling_on_sc=False),
    )
    def k(table_hbm, src_hbm, dst_hbm, zeros_hbm, out_hbm,
          idx_s, idx_d, rows0, rows1, acc, sem0, sem1, semz):
        c = lax.axis_index("c")
        s = lax.axis_index("s")
        wid = c * NS + s

        zero = pltpu.async_copy(zeros_hbm, acc.at[pl.ds(s * RPT, RPT)], semz)
        pltpu.sync_copy(dst_hbm.at[wid], idx_d)
        if table_is_ones:
            pltpu.sync_copy(table_hbm, rows0)
        else:
            pltpu.sync_copy(src_hbm.at[wid], idx_s)
        zero.wait()
        plsc.subcore_barrier()

        if table_is_ones:
            G = 8
            def body(i, carry):
                for b in range(G):
                    pltpu.async_copy(rows0, acc.at[idx_d.at[i * G + b]],
                                     sem0, add=True)
                for b in range(G):
                    pltpu.make_async_copy(
                        rows0, acc.at[idx_d.at[i * G + b]], sem0).wait()
                return carry
            lax.fori_loop(0, NCH // G, body, 0)
        else:
            pltpu.async_copy(table_hbm.at[idx_s.at[0]], rows0, sem0)

            def body(i, carry):
                j0 = 2 * i
                j1 = j0 + 1
                pltpu.async_copy(table_hbm.at[idx_s.at[j1]], rows1, sem1)
                pltpu.make_async_copy(
                    table_hbm.at[idx_s.at[j0]], rows0, sem0).wait()
                pltpu.sync_copy(rows0, acc.at[idx_d.at[j0]], add=True)

                @pl.when(i < NCH // 2 - 1)
                def _next():
                    pltpu.async_copy(
                        table_hbm.at[idx_s.at[j0 + 2]], rows0, sem0)
                pltpu.make_async_copy(
                    table_hbm.at[idx_s.at[j1]], rows1, sem1).wait()
                pltpu.sync_copy(rows1, acc.at[idx_d.at[j1]], add=True)
                return carry

            lax.fori_loop(0, NCH // 2, body, 0)

        plsc.subcore_barrier()
        pltpu.sync_copy(acc.at[pl.ds(s * RPT, RPT)],
                        out_hbm.at[c, pl.ds(s * RPT, RPT)])

    return k


_seg_agg = {D: _make_seg_agg(D, False) for D in (64, 32)}
_deg_agg = _make_seg_agg(1, True)


def _mm_body(x_ref, w_ref, o_ref):
    o_ref[...] = jnp.dot(x_ref[...], w_ref[...],
                         preferred_element_type=jnp.float32)


def _scale_body(h_ref, degp_ref, g_ref, dis_ref):
    deg = degp_ref[0, :N_NODES, :] + degp_ref[1, :N_NODES, :] + 1.0
    dis = lax.rsqrt(deg)
    dis_ref[...] = dis
    g_ref[...] = h_ref[...] * dis


def _layer_body(accp_ref, g_ref, dis_ref, b_ref, w_ref, g2_ref):
    dis = dis_ref[...]
    h = dis * (accp_ref[0, :N_NODES] + accp_ref[1, :N_NODES] + g_ref[...]) + b_ref[...]
    h = jnp.maximum(h, 0.0)
    g2_ref[...] = dis * jnp.dot(h, w_ref[...],
                                preferred_element_type=jnp.float32)


def _final_body(accp_ref, g_ref, dis_ref, b_ref, wat_ref, ba_ref, o_ref):
    dis = dis_ref[...]
    h = dis * (accp_ref[0, :N_NODES] + accp_ref[1, :N_NODES] + g_ref[...]) + b_ref[...]
    h = jnp.maximum(h, 0.0)
    logit = jnp.sum(h * wat_ref[...], axis=1, keepdims=True) + ba_ref[...]
    attn = 1.0 / (1.0 + jnp.exp(-logit))
    o_ref[...] = h * attn


def _tc(body, out_shape, *args):
    return pl.pallas_call(
        body, out_shape=jax.ShapeDtypeStruct(out_shape, jnp.float32))(*args)


def kernel(x, edge_index, W1, b1, W2, b2, Wa, ba):
    src = edge_index[0].astype(jnp.int32).reshape(NW, NCH, CH)
    dst = edge_index[1].astype(jnp.int32).reshape(NW, NCH, CH)

    ones_rows = jnp.ones((CH, 1), jnp.float32)
    zeros_rpt1 = jnp.zeros((RPT, 1), jnp.float32)
    zeros_rpt64 = jnp.zeros((RPT, 64), jnp.float32)
    zeros_rpt32 = jnp.zeros((RPT, 32), jnp.float32)

    h1p = _tc(_mm_body, (N_NODES, 64), x, W1)
    degp = _deg_agg(ones_rows, src, dst, zeros_rpt1)

    g1, dis = pl.pallas_call(
        _scale_body,
        out_shape=(jax.ShapeDtypeStruct((N_NODES, 64), jnp.float32),
                   jax.ShapeDtypeStruct((N_NODES, 1), jnp.float32)),
    )(h1p, degp)

    acc1 = _seg_agg[64](g1, src, dst, zeros_rpt64)
    g2 = _tc(_layer_body, (N_NODES, 32),
             acc1, g1, dis, b1.reshape(1, 64), W2)

    acc2 = _seg_agg[32](g2, src, dst, zeros_rpt32)
    out = _tc(_final_body, (N_NODES, 32),
              acc2, g2, dis, b2.reshape(1, 32), Wa.reshape(1, 32),
              ba.reshape(1, 1))
    return out


# speedup vs baseline: 45.0591x; 1.1021x over previous
import functools

import jax
import jax.numpy as jnp
from jax import lax
from jax.experimental import pallas as pl
from jax.experimental.pallas import tpu as pltpu
from jax.experimental.pallas import tpu_sc as plsc

N_NODES = 10000
N_EDGES = 320000
NC = 2
NS = 16
NW = NC * NS
E_PER_TILE = N_EDGES // NW
CH = 250
NCH = E_PER_TILE // CH
RPT = 632
N_PAD = RPT * NS


def _make_seg_agg(D, table_is_ones):
    mesh = plsc.VectorSubcoreMesh(core_axis_name="c", subcore_axis_name="s")

    scratch = [
        pltpu.VMEM((NCH, CH), jnp.int32),
        pltpu.VMEM((NCH, CH), jnp.int32),
        pltpu.VMEM((CH, D), jnp.float32),
        pltpu.VMEM((CH, D), jnp.float32),
        pltpu.VMEM_SHARED((N_PAD, D), jnp.float32),
        pltpu.SemaphoreType.DMA,
        pltpu.SemaphoreType.DMA,
        pltpu.SemaphoreType.DMA,
    ]

    @functools.partial(
        pl.kernel,
        out_type=jax.ShapeDtypeStruct((NC, N_PAD, D), jnp.float32),
        mesh=mesh,
        scratch_types=scratch,
        compiler_params=pltpu.CompilerParams(use_tc_tiling_on_sc=False),
    )
    def k(table_hbm, src_hbm, dst_hbm, zeros_hbm, out_hbm,
          idx_s, idx_d, rows0, rows1, acc, sem0, sem1, semz):
        c = lax.axis_index("c")
        s = lax.axis_index("s")
        wid = c * NS + s

        zero = pltpu.async_copy(zeros_hbm, acc.at[pl.ds(s * RPT, RPT)], semz)
        pltpu.sync_copy(dst_hbm.at[wid], idx_d)
        if table_is_ones:
            pltpu.sync_copy(table_hbm, rows0)
        else:
            pltpu.sync_copy(src_hbm.at[wid], idx_s)
        zero.wait()
        plsc.subcore_barrier()

        if table_is_ones:
            G = 8
            def body(i, carry):
                for b in range(G):
                    pltpu.async_copy(rows0, acc.at[idx_d.at[i * G + b]],
                                     sem0, add=True)
                for b in range(G):
                    pltpu.make_async_copy(
                        rows0, acc.at[idx_d.at[i * G + b]], sem0).wait()
                return carry
            lax.fori_loop(0, NCH // G, body, 0)
        else:
            pltpu.async_copy(table_hbm.at[idx_s.at[0]], rows0, sem0)

            def body(i, carry):
                j0 = 2 * i
                j1 = j0 + 1
                pltpu.async_copy(table_hbm.at[idx_s.at[j1]], rows1, sem1)
                pltpu.make_async_copy(
                    table_hbm.at[idx_s.at[j0]], rows0, sem0).wait()
                pltpu.sync_copy(rows0, acc.at[idx_d.at[j0]], add=True)

                @pl.when(i < NCH // 2 - 1)
                def _next():
                    pltpu.async_copy(
                        table_hbm.at[idx_s.at[j0 + 2]], rows0, sem0)
                pltpu.make_async_copy(
                    table_hbm.at[idx_s.at[j1]], rows1, sem1).wait()
                pltpu.sync_copy(rows1, acc.at[idx_d.at[j1]], add=True)
                return carry

            lax.fori_loop(0, NCH // 2, body, 0)

        plsc.subcore_barrier()
        pltpu.sync_copy(acc.at[pl.ds(s * RPT, RPT)],
                        out_hbm.at[c, pl.ds(s * RPT, RPT)])

    return k


_seg_agg = {D: _make_seg_agg(D, False) for D in (64, 32)}
_deg_agg = _make_seg_agg(1, True)


def _mm_body(x_ref, w_ref, o_ref):
    o_ref[...] = jnp.dot(x_ref[...], w_ref[...],
                         preferred_element_type=jnp.float32)


def _scale_body(h_ref, degp_ref, g_ref, dis_ref):
    deg = degp_ref[0, :N_NODES, :] + degp_ref[1, :N_NODES, :] + 1.0
    dis = lax.rsqrt(deg)
    dis_ref[...] = dis
    g_ref[...] = h_ref[...] * dis


def _layer_body(accp_ref, g_ref, dis_ref, b_ref, w_ref, g2_ref):
    dis = dis_ref[...]
    h = dis * (accp_ref[0, :N_NODES] + accp_ref[1, :N_NODES] + g_ref[...]) + b_ref[...]
    h = jnp.maximum(h, 0.0)
    g2_ref[...] = dis * jnp.dot(h, w_ref[...],
                                preferred_element_type=jnp.float32)


def _final_body(accp_ref, g_ref, dis_ref, b_ref, wat_ref, ba_ref, o_ref):
    dis = dis_ref[...]
    h = dis * (accp_ref[0, :N_NODES] + accp_ref[1, :N_NODES] + g_ref[...]) + b_ref[...]
    h = jnp.maximum(h, 0.0)
    logit = jnp.sum(h * wat_ref[...], axis=1, keepdims=True) + ba_ref[...]
    attn = 1.0 / (1.0 + jnp.exp(-logit))
    o_ref[...] = h * attn


def _tc(body, out_shape, *args):
    return pl.pallas_call(
        body, out_shape=jax.ShapeDtypeStruct(out_shape, jnp.float32))(*args)


def kernel(x, edge_index, W1, b1, W2, b2, Wa, ba):
    src = edge_index[0].astype(jnp.int32).reshape(NW, NCH, CH)
    dst = edge_index[1].astype(jnp.int32).reshape(NW, NCH, CH)

    ones_rows = jnp.ones((CH, 1), jnp.float32)
    zeros_rpt1 = jnp.zeros((RPT, 1), jnp.float32)
    zeros_rpt64 = jnp.zeros((RPT, 64), jnp.float32)
    zeros_rpt32 = jnp.zeros((RPT, 32), jnp.float32)

    h1p = _tc(_mm_body, (N_NODES, 64), x, W1)
    degp = _deg_agg(ones_rows, src, dst, zeros_rpt1)

    g1, dis = pl.pallas_call(
        _scale_body,
        out_shape=(jax.ShapeDtypeStruct((N_NODES, 64), jnp.float32),
                   jax.ShapeDtypeStruct((N_NODES, 1), jnp.float32)),
    )(h1p, degp)

    acc1 = _seg_agg[64](g1, src, dst, zeros_rpt64)
    g2 = _tc(_layer_body, (N_NODES, 32),
             acc1, g1, dis, b1.reshape(1, 64), W2)

    acc2 = _seg_agg[32](g2, src, dst, zeros_rpt32)
    out = _tc(_final_body, (N_NODES, 32),
              acc2, g2, dis, b2.reshape(1, 32), Wa.reshape(1, 32),
              ba.reshape(1, 1))
    return out


# speedup vs baseline: 45.7831x; 1.0161x over previous
import functools

import jax
import jax.numpy as jnp
from jax import lax
from jax.experimental import pallas as pl
from jax.experimental.pallas import tpu as pltpu
from jax.experimental.pallas import tpu_sc as plsc

N_NODES = 10000
N_EDGES = 320000
NC = 2
NS = 16
NW = NC * NS
E_PER_TILE = N_EDGES // NW
CH = 500
NCH = E_PER_TILE // CH
RPT = 632
N_PAD = RPT * NS


def _make_seg_agg(D, table_is_ones):
    mesh = plsc.VectorSubcoreMesh(core_axis_name="c", subcore_axis_name="s")

    scratch = [
        pltpu.VMEM((NCH, CH), jnp.int32),
        pltpu.VMEM((NCH, CH), jnp.int32),
        pltpu.VMEM((CH, D), jnp.float32),
        pltpu.VMEM((CH, D), jnp.float32),
        pltpu.VMEM_SHARED((N_PAD, D), jnp.float32),
        pltpu.SemaphoreType.DMA,
        pltpu.SemaphoreType.DMA,
        pltpu.SemaphoreType.DMA,
    ]

    @functools.partial(
        pl.kernel,
        out_type=jax.ShapeDtypeStruct((NC, N_PAD, D), jnp.float32),
        mesh=mesh,
        scratch_types=scratch,
        compiler_params=pltpu.CompilerParams(use_tc_tiling_on_sc=False),
    )
    def k(table_hbm, src_hbm, dst_hbm, zeros_hbm, out_hbm,
          idx_s, idx_d, rows0, rows1, acc, sem0, sem1, semz):
        c = lax.axis_index("c")
        s = lax.axis_index("s")
        wid = c * NS + s

        zero = pltpu.async_copy(zeros_hbm, acc.at[pl.ds(s * RPT, RPT)], semz)
        pltpu.sync_copy(dst_hbm.at[wid], idx_d)
        if table_is_ones:
            pltpu.sync_copy(table_hbm, rows0)
        else:
            pltpu.sync_copy(src_hbm.at[wid], idx_s)
        zero.wait()
        plsc.subcore_barrier()

        if table_is_ones:
            G = 10
            def body(i, carry):
                for b in range(G):
                    pltpu.async_copy(rows0, acc.at[idx_d.at[i * G + b]],
                                     sem0, add=True)
                for b in range(G):
                    pltpu.make_async_copy(
                        rows0, acc.at[idx_d.at[i * G + b]], sem0).wait()
                return carry
            lax.fori_loop(0, NCH // G, body, 0)
        else:
            pltpu.async_copy(table_hbm.at[idx_s.at[0]], rows0, sem0)

            def body(i, carry):
                j0 = 2 * i
                j1 = j0 + 1
                pltpu.async_copy(table_hbm.at[idx_s.at[j1]], rows1, sem1)
                pltpu.make_async_copy(
                    table_hbm.at[idx_s.at[j0]], rows0, sem0).wait()
                pltpu.sync_copy(rows0, acc.at[idx_d.at[j0]], add=True)

                @pl.when(i < NCH // 2 - 1)
                def _next():
                    pltpu.async_copy(
                        table_hbm.at[idx_s.at[j0 + 2]], rows0, sem0)
                pltpu.make_async_copy(
                    table_hbm.at[idx_s.at[j1]], rows1, sem1).wait()
                pltpu.sync_copy(rows1, acc.at[idx_d.at[j1]], add=True)
                return carry

            lax.fori_loop(0, NCH // 2, body, 0)

        plsc.subcore_barrier()
        pltpu.sync_copy(acc.at[pl.ds(s * RPT, RPT)],
                        out_hbm.at[c, pl.ds(s * RPT, RPT)])

    return k


_seg_agg = {D: _make_seg_agg(D, False) for D in (64, 32)}
_deg_agg = _make_seg_agg(1, True)


def _mm_body(x_ref, w_ref, o_ref):
    o_ref[...] = jnp.dot(x_ref[...], w_ref[...],
                         preferred_element_type=jnp.float32)


def _scale_body(h_ref, degp_ref, g_ref, dis_ref):
    deg = degp_ref[0, :N_NODES, :] + degp_ref[1, :N_NODES, :] + 1.0
    dis = lax.rsqrt(deg)
    dis_ref[...] = dis
    g_ref[...] = h_ref[...] * dis


def _layer_body(accp_ref, g_ref, dis_ref, b_ref, w_ref, g2_ref):
    dis = dis_ref[...]
    h = dis * (accp_ref[0, :N_NODES] + accp_ref[1, :N_NODES] + g_ref[...]) + b_ref[...]
    h = jnp.maximum(h, 0.0)
    g2_ref[...] = dis * jnp.dot(h, w_ref[...],
                                preferred_element_type=jnp.float32)


def _final_body(accp_ref, g_ref, dis_ref, b_ref, wat_ref, ba_ref, o_ref):
    dis = dis_ref[...]
    h = dis * (accp_ref[0, :N_NODES] + accp_ref[1, :N_NODES] + g_ref[...]) + b_ref[...]
    h = jnp.maximum(h, 0.0)
    logit = jnp.sum(h * wat_ref[...], axis=1, keepdims=True) + ba_ref[...]
    attn = 1.0 / (1.0 + jnp.exp(-logit))
    o_ref[...] = h * attn


def _tc(body, out_shape, *args):
    return pl.pallas_call(
        body, out_shape=jax.ShapeDtypeStruct(out_shape, jnp.float32))(*args)


def kernel(x, edge_index, W1, b1, W2, b2, Wa, ba):
    src = edge_index[0].astype(jnp.int32).reshape(NW, NCH, CH)
    dst = edge_index[1].astype(jnp.int32).reshape(NW, NCH, CH)

    ones_rows = jnp.ones((CH, 1), jnp.float32)
    zeros_rpt1 = jnp.zeros((RPT, 1), jnp.float32)
    zeros_rpt64 = jnp.zeros((RPT, 64), jnp.float32)
    zeros_rpt32 = jnp.zeros((RPT, 32), jnp.float32)

    h1p = _tc(_mm_body, (N_NODES, 64), x, W1)
    degp = _deg_agg(ones_rows, src, dst, zeros_rpt1)

    g1, dis = pl.pallas_call(
        _scale_body,
        out_shape=(jax.ShapeDtypeStruct((N_NODES, 64), jnp.float32),
                   jax.ShapeDtypeStruct((N_NODES, 1), jnp.float32)),
    )(h1p, degp)

    acc1 = _seg_agg[64](g1, src, dst, zeros_rpt64)
    g2 = _tc(_layer_body, (N_NODES, 32),
             acc1, g1, dis, b1.reshape(1, 64), W2)

    acc2 = _seg_agg[32](g2, src, dst, zeros_rpt32)
    out = _tc(_final_body, (N_NODES, 32),
              acc2, g2, dis, b2.reshape(1, 32), Wa.reshape(1, 32),
              ba.reshape(1, 1))
    return out


# speedup vs baseline: 47.5793x; 1.0392x over previous
import functools

import jax
import jax.numpy as jnp
from jax import lax
from jax.experimental import pallas as pl
from jax.experimental.pallas import tpu as pltpu
from jax.experimental.pallas import tpu_sc as plsc

N_NODES = 10000
N_EDGES = 320000
NC = 2
NS = 16
NW = NC * NS
E_PER_TILE = N_EDGES // NW
CH = 250
NB = 4
NCH = E_PER_TILE // CH
RPT = 632
N_PAD = RPT * NS


def _make_seg_agg(D, table_is_ones):
    mesh = plsc.VectorSubcoreMesh(core_axis_name="c", subcore_axis_name="s")

    scratch = (
        [pltpu.VMEM((NCH, CH), jnp.int32)] * 2
        + [pltpu.VMEM((CH, D), jnp.float32)] * NB
        + [pltpu.VMEM_SHARED((N_PAD, D), jnp.float32)]
        + [pltpu.SemaphoreType.DMA] * (NB + 1)
    )

    @functools.partial(
        pl.kernel,
        out_type=jax.ShapeDtypeStruct((NC, N_PAD, D), jnp.float32),
        mesh=mesh,
        scratch_types=scratch,
        compiler_params=pltpu.CompilerParams(use_tc_tiling_on_sc=False),
    )
    def k(table_hbm, src_hbm, dst_hbm, zeros_hbm, out_hbm,
          idx_s, idx_d, *bufs):
        rows = list(bufs[:NB])
        acc = bufs[NB]
        semg = list(bufs[NB + 1:NB + 1 + NB])
        semz = bufs[NB + 1 + NB]
        rows0 = rows[0]
        sem0 = semg[0]
        c = lax.axis_index("c")
        s = lax.axis_index("s")
        wid = c * NS + s

        zero = pltpu.async_copy(zeros_hbm, acc.at[pl.ds(s * RPT, RPT)], semz)
        pltpu.sync_copy(dst_hbm.at[wid], idx_d)
        if table_is_ones:
            pltpu.sync_copy(table_hbm, rows0)
        else:
            pltpu.sync_copy(src_hbm.at[wid], idx_s)
        zero.wait()
        plsc.subcore_barrier()

        if table_is_ones:
            G = 10
            def body(i, carry):
                for b in range(G):
                    pltpu.async_copy(rows0, acc.at[idx_d.at[i * G + b]],
                                     sem0, add=True)
                for b in range(G):
                    pltpu.make_async_copy(
                        rows0, acc.at[idx_d.at[i * G + b]], sem0).wait()
                return carry
            lax.fori_loop(0, NCH // G, body, 0)
        else:
            for b in range(NB):
                pltpu.async_copy(table_hbm.at[idx_s.at[b]], rows[b], semg[b])

            def body(i, carry):
                for b in range(NB):
                    j = i * NB + b
                    pltpu.make_async_copy(
                        table_hbm.at[idx_s.at[j]], rows[b], semg[b]).wait()
                    pltpu.sync_copy(rows[b], acc.at[idx_d.at[j]], add=True)

                    @pl.when(j + NB < NCH)
                    def _next(b=b, j=j):
                        pltpu.async_copy(
                            table_hbm.at[idx_s.at[j + NB]], rows[b], semg[b])
                return carry

            lax.fori_loop(0, NCH // NB, body, 0)

        plsc.subcore_barrier()
        pltpu.sync_copy(acc.at[pl.ds(s * RPT, RPT)],
                        out_hbm.at[c, pl.ds(s * RPT, RPT)])

    return k


_seg_agg = {D: _make_seg_agg(D, False) for D in (64, 32)}
_deg_agg = _make_seg_agg(1, True)


def _mm_body(x_ref, w_ref, o_ref):
    o_ref[...] = jnp.dot(x_ref[...], w_ref[...],
                         preferred_element_type=jnp.float32)


def _scale_body(h_ref, degp_ref, g_ref, dis_ref):
    deg = degp_ref[0, :N_NODES, :] + degp_ref[1, :N_NODES, :] + 1.0
    dis = lax.rsqrt(deg)
    dis_ref[...] = dis
    g_ref[...] = h_ref[...] * dis


def _layer_body(accp_ref, g_ref, dis_ref, b_ref, w_ref, g2_ref):
    dis = dis_ref[...]
    h = dis * (accp_ref[0, :N_NODES] + accp_ref[1, :N_NODES] + g_ref[...]) + b_ref[...]
    h = jnp.maximum(h, 0.0)
    g2_ref[...] = dis * jnp.dot(h, w_ref[...],
                                preferred_element_type=jnp.float32)


def _final_body(accp_ref, g_ref, dis_ref, b_ref, wat_ref, ba_ref, o_ref):
    dis = dis_ref[...]
    h = dis * (accp_ref[0, :N_NODES] + accp_ref[1, :N_NODES] + g_ref[...]) + b_ref[...]
    h = jnp.maximum(h, 0.0)
    logit = jnp.sum(h * wat_ref[...], axis=1, keepdims=True) + ba_ref[...]
    attn = 1.0 / (1.0 + jnp.exp(-logit))
    o_ref[...] = h * attn


def _tc(body, out_shape, *args):
    return pl.pallas_call(
        body, out_shape=jax.ShapeDtypeStruct(out_shape, jnp.float32))(*args)


def kernel(x, edge_index, W1, b1, W2, b2, Wa, ba):
    src = edge_index[0].astype(jnp.int32).reshape(NW, NCH, CH)
    dst = edge_index[1].astype(jnp.int32).reshape(NW, NCH, CH)

    ones_rows = jnp.ones((CH, 1), jnp.float32)
    zeros_rpt1 = jnp.zeros((RPT, 1), jnp.float32)
    zeros_rpt64 = jnp.zeros((RPT, 64), jnp.float32)
    zeros_rpt32 = jnp.zeros((RPT, 32), jnp.float32)

    h1p = _tc(_mm_body, (N_NODES, 64), x, W1)
    degp = _deg_agg(ones_rows, src, dst, zeros_rpt1)

    g1, dis = pl.pallas_call(
        _scale_body,
        out_shape=(jax.ShapeDtypeStruct((N_NODES, 64), jnp.float32),
                   jax.ShapeDtypeStruct((N_NODES, 1), jnp.float32)),
    )(h1p, degp)

    acc1 = _seg_agg[64](g1, src, dst, zeros_rpt64)
    g2 = _tc(_layer_body, (N_NODES, 32),
             acc1, g1, dis, b1.reshape(1, 64), W2)

    acc2 = _seg_agg[32](g2, src, dst, zeros_rpt32)
    out = _tc(_final_body, (N_NODES, 32),
              acc2, g2, dis, b2.reshape(1, 32), Wa.reshape(1, 32),
              ba.reshape(1, 1))
    return out
